# Initial kernel scaffold; baseline (speedup 1.0000x reference)
import jax
import jax.numpy as jnp
from jax.experimental import pallas as pl


def kernel(A_stack, lnc_sim, dis_sim, miR_sim, W_l, b_l, W_d, b_d, W_m, b_m, weight):
    raise NotImplementedError("write your pallas kernel here")


# speedup vs baseline: 1.0776x; 1.0776x over previous
import jax
import jax.numpy as jnp
from jax.experimental import pallas as pl
from jax.experimental.pallas import tpu as pltpu

_EPS = 1e-8
_LAT = 128


def _ego_body(sim_ref, w_ref, b_ref, e_ref, n_ref, acc):
    k = pl.program_id(1)

    @pl.when(k == 0)
    def _init():
        acc[...] = jnp.zeros_like(acc)

    acc[...] += jax.lax.dot_general(
        sim_ref[...].astype(jnp.bfloat16), w_ref[...].astype(jnp.bfloat16),
        (((1,), (0,)), ((), ())), preferred_element_type=jnp.float32)

    @pl.when(k == pl.num_programs(1) - 1)
    def _fin():
        e = jax.nn.sigmoid(acc[...] + b_ref[...])
        e_ref[...] = e
        n_ref[...] = jnp.maximum(
            jnp.sqrt(jnp.sum(e * e, axis=1, keepdims=True)), _EPS)


def _ego_call(sim, w, b, bm, bk):
    r = sim.shape[0]
    grid = (r // bm, r // bk)
    return pl.pallas_call(
        _ego_body,
        grid=grid,
        in_specs=[
            pl.BlockSpec((bm, bk), lambda i, k: (i, k)),
            pl.BlockSpec((bk, _LAT), lambda i, k: (k, 0)),
            pl.BlockSpec((1, _LAT), lambda i, k: (0, 0)),
        ],
        out_specs=[
            pl.BlockSpec((bm, _LAT), lambda i, k: (i, 0)),
            pl.BlockSpec((bm, 1), lambda i, k: (i, 0)),
        ],
        out_shape=[
            jax.ShapeDtypeStruct((r, _LAT), jnp.float32),
            jax.ShapeDtypeStruct((r, 1), jnp.float32),
        ],
        scratch_shapes=[pltpu.VMEM((bm, _LAT), jnp.float32)],
    )(sim, w, b.reshape(1, _LAT))


def _prep_body(a_ref, ab_ref, d_ref, deg):
    k = pl.program_id(1)
    blk = a_ref[...]
    ab_ref[...] = blk.astype(jnp.bfloat16)

    @pl.when(k == 0)
    def _init():
        deg[...] = jnp.zeros_like(deg)

    deg[...] += jnp.sum(blk, axis=1, keepdims=True)

    @pl.when(k == pl.num_programs(1) - 1)
    def _fin():
        dg = deg[...]
        d_ref[...] = jnp.where(dg > 0, jax.lax.rsqrt(jnp.maximum(dg, _EPS)),
                               0.0)


def _prep_call(a, bm, bk):
    n = a.shape[0]
    grid = (n // bm, n // bk)
    return pl.pallas_call(
        _prep_body,
        grid=grid,
        in_specs=[pl.BlockSpec((bm, bk), lambda i, k: (i, k))],
        out_specs=[
            pl.BlockSpec((bm, bk), lambda i, k: (i, k)),
            pl.BlockSpec((bm, 1), lambda i, k: (i, 0)),
        ],
        out_shape=[
            jax.ShapeDtypeStruct((n, n), jnp.bfloat16),
            jax.ShapeDtypeStruct((n, 1), jnp.float32),
        ],
        scratch_shapes=[pltpu.VMEM((bm, 1), jnp.float32)],
    )(a)


def _g0_body(ego_ref, d_ref, w_ref, g_ref):
    g_ref[...] = jax.lax.dot_general(
        (d_ref[...] * ego_ref[...]).astype(jnp.bfloat16),
        w_ref[...].astype(jnp.bfloat16),
        (((1,), (0,)), ((), ())),
        preferred_element_type=jnp.float32).astype(jnp.bfloat16)


def _g0_call(ego, d, w, bm):
    n = ego.shape[0]
    return pl.pallas_call(
        _g0_body,
        grid=(n // bm,),
        in_specs=[
            pl.BlockSpec((bm, _LAT), lambda i: (i, 0)),
            pl.BlockSpec((bm, 1), lambda i: (i, 0)),
            pl.BlockSpec((_LAT, _LAT), lambda i: (0, 0)),
        ],
        out_specs=pl.BlockSpec((bm, _LAT), lambda i: (i, 0)),
        out_shape=jax.ShapeDtypeStruct((n, _LAT), jnp.bfloat16),
    )(ego, d, w)


def _layer_body(a_ref, g_ref, d_ref, ego_ref, en_ref, w_ref,
                contrib_ref, gnext_ref, acc):
    k = pl.program_id(1)

    @pl.when(k == 0)
    def _init():
        acc[...] = jnp.zeros_like(acc)

    acc[...] += jax.lax.dot_general(
        a_ref[...], g_ref[...], (((1,), (0,)), ((), ())),
        preferred_element_type=jnp.float32)

    @pl.when(k == pl.num_programs(1) - 1)
    def _fin():
        d = d_ref[...]
        lay = jnp.maximum(d * acc[...], 0.0)
        ln = jnp.maximum(jnp.sqrt(jnp.sum(lay * lay, axis=1, keepdims=True)),
                         _EPS)
        wgt = jnp.sum(lay * ego_ref[...], axis=1, keepdims=True) / (
            ln * en_ref[...])
        contrib_ref[...] = wgt * lay
        gnext_ref[...] = jax.lax.dot_general(
            (d * lay).astype(jnp.bfloat16), w_ref[...].astype(jnp.bfloat16),
            (((1,), (0,)), ((), ())),
            preferred_element_type=jnp.float32).astype(jnp.bfloat16)


def _layer_call(a_bf, g, d, ego, en, w, bm, bk):
    n = a_bf.shape[0]
    grid = (n // bm, n // bk)
    return pl.pallas_call(
        _layer_body,
        grid=grid,
        in_specs=[
            pl.BlockSpec((bm, bk), lambda i, k: (i, k)),
            pl.BlockSpec((bk, _LAT), lambda i, k: (k, 0)),
            pl.BlockSpec((bm, 1), lambda i, k: (i, 0)),
            pl.BlockSpec((bm, _LAT), lambda i, k: (i, 0)),
            pl.BlockSpec((bm, 1), lambda i, k: (i, 0)),
            pl.BlockSpec((_LAT, _LAT), lambda i, k: (0, 0)),
        ],
        out_specs=[
            pl.BlockSpec((bm, _LAT), lambda i, k: (i, 0)),
            pl.BlockSpec((bm, _LAT), lambda i, k: (i, 0)),
        ],
        out_shape=[
            jax.ShapeDtypeStruct((n, _LAT), jnp.float32),
            jax.ShapeDtypeStruct((n, _LAT), jnp.bfloat16),
        ],
        scratch_shapes=[pltpu.VMEM((bm, _LAT), jnp.float32)],
    )(a_bf, g, d, ego, en, w)


def _pred_body(l1, l2, l3, d1, d2, d3, out_ref):
    lm = ((l1[...] + l2[...] + l3[...]) * (1.0 / 3.0)).astype(jnp.bfloat16)
    dm = ((d1[...] + d2[...] + d3[...]) * (1.0 / 3.0)).astype(jnp.bfloat16)
    out_ref[...] = jax.lax.dot_general(
        lm, dm, (((1,), (1,)), ((), ())), preferred_element_type=jnp.float32)


def _pred_call(ls, ds, bm):
    lr = ls[0].shape[0]
    dr = ds[0].shape[0]
    return pl.pallas_call(
        _pred_body,
        grid=(lr // bm,),
        in_specs=[pl.BlockSpec((bm, _LAT), lambda i: (i, 0))] * 3
        + [pl.BlockSpec((dr, _LAT), lambda i: (0, 0))] * 3,
        out_specs=pl.BlockSpec((bm, dr), lambda i: (i, 0)),
        out_shape=jax.ShapeDtypeStruct((lr, dr), jnp.float32),
    )(*ls, *ds)


def kernel(A_stack, lnc_sim, dis_sim, miR_sim, W_l, b_l, W_d, b_d, W_m, b_m,
           weight):
    l_num = lnc_sim.shape[0]
    d_num = dis_sim.shape[0]
    n = A_stack.shape[0]

    def ego(sim, w, b):
        r = sim.shape[0]
        return _ego_call(sim, w, b, bm=min(1024, r), bk=min(2048, r))

    e_l, n_l = ego(lnc_sim, W_l, b_l)
    e_d, n_d = ego(dis_sim, W_d, b_d)
    e_m, n_m = ego(miR_sim, W_m, b_m)
    ego_all = jnp.concatenate([e_l, e_d, e_m], axis=0)
    en = jnp.concatenate([n_l, n_d, n_m], axis=0)

    a_bf, d = _prep_call(A_stack, bm=min(1024, n), bk=min(2048, n))

    g = _g0_call(ego_all, d, weight, bm=min(2048, n))
    contribs = []
    for _ in range(3):
        contrib, g = _layer_call(a_bf, g, d, ego_all, en, weight,
                                 bm=min(2048, n), bk=min(2048, n))
        contribs.append(contrib)

    ls = [c[:l_num] for c in contribs]
    ds = [c[l_num:l_num + d_num] for c in contribs]
    return _pred_call(ls, ds, bm=min(1024, l_num))


# speedup vs baseline: 1.0826x; 1.0047x over previous
import jax
import jax.numpy as jnp
from jax.experimental import pallas as pl
from jax.experimental.pallas import tpu as pltpu

_EPS = 1e-8
_LAT = 128


def _ego_body(sim_ref, w_ref, b_ref, e_ref, n_ref):
    acc = jax.lax.dot_general(
        sim_ref[...].astype(jnp.bfloat16), w_ref[...].astype(jnp.bfloat16),
        (((1,), (0,)), ((), ())), preferred_element_type=jnp.float32)
    e = jax.nn.sigmoid(acc + b_ref[...])
    e_ref[...] = e
    n_ref[...] = jnp.maximum(
        jnp.sqrt(jnp.sum(e * e, axis=1, keepdims=True)), _EPS)


def _ego_call(sim, w, b, bm):
    r = sim.shape[0]
    return pl.pallas_call(
        _ego_body,
        grid=(r // bm,),
        in_specs=[
            pl.BlockSpec((bm, r), lambda i: (i, 0)),
            pl.BlockSpec((r, _LAT), lambda i: (0, 0)),
            pl.BlockSpec((1, _LAT), lambda i: (0, 0)),
        ],
        out_specs=[
            pl.BlockSpec((bm, _LAT), lambda i: (i, 0)),
            pl.BlockSpec((bm, 1), lambda i: (i, 0)),
        ],
        out_shape=[
            jax.ShapeDtypeStruct((r, _LAT), jnp.float32),
            jax.ShapeDtypeStruct((r, 1), jnp.float32),
        ],
    )(sim, w, b.reshape(1, _LAT))


def _prep_body(a_ref, ab_ref, d_ref):
    blk = a_ref[...]
    ab_ref[...] = blk.astype(jnp.bfloat16)
    dg = jnp.sum(blk, axis=1, keepdims=True)
    d_ref[...] = jnp.where(dg > 0, jax.lax.rsqrt(jnp.maximum(dg, _EPS)), 0.0)


def _prep_call(a, bm):
    n = a.shape[0]
    return pl.pallas_call(
        _prep_body,
        grid=(n // bm,),
        in_specs=[pl.BlockSpec((bm, n), lambda i: (i, 0))],
        out_specs=[
            pl.BlockSpec((bm, n), lambda i: (i, 0)),
            pl.BlockSpec((bm, 1), lambda i: (i, 0)),
        ],
        out_shape=[
            jax.ShapeDtypeStruct((n, n), jnp.bfloat16),
            jax.ShapeDtypeStruct((n, 1), jnp.float32),
        ],
    )(a)


def _g0_body(ego_ref, d_ref, w_ref, g_ref):
    g_ref[...] = jax.lax.dot_general(
        (d_ref[...] * ego_ref[...]).astype(jnp.bfloat16),
        w_ref[...].astype(jnp.bfloat16),
        (((1,), (0,)), ((), ())),
        preferred_element_type=jnp.float32).astype(jnp.bfloat16)


def _g0_call(ego, d, w, bm):
    n = ego.shape[0]
    return pl.pallas_call(
        _g0_body,
        grid=(n // bm,),
        in_specs=[
            pl.BlockSpec((bm, _LAT), lambda i: (i, 0)),
            pl.BlockSpec((bm, 1), lambda i: (i, 0)),
            pl.BlockSpec((_LAT, _LAT), lambda i: (0, 0)),
        ],
        out_specs=pl.BlockSpec((bm, _LAT), lambda i: (i, 0)),
        out_shape=jax.ShapeDtypeStruct((n, _LAT), jnp.bfloat16),
    )(ego, d, w)


def _layer_body(a_ref, g_ref, d_ref, ego_ref, en_ref, w_ref,
                contrib_ref, gnext_ref):
    acc = jax.lax.dot_general(
        a_ref[...], g_ref[...], (((1,), (0,)), ((), ())),
        preferred_element_type=jnp.float32)
    d = d_ref[...]
    lay = jnp.maximum(d * acc, 0.0)
    ln = jnp.maximum(jnp.sqrt(jnp.sum(lay * lay, axis=1, keepdims=True)),
                     _EPS)
    wgt = jnp.sum(lay * ego_ref[...], axis=1, keepdims=True) / (
        ln * en_ref[...])
    contrib_ref[...] = wgt * lay
    gnext_ref[...] = jax.lax.dot_general(
        (d * lay).astype(jnp.bfloat16), w_ref[...].astype(jnp.bfloat16),
        (((1,), (0,)), ((), ())),
        preferred_element_type=jnp.float32).astype(jnp.bfloat16)


def _layer_call(a_bf, g, d, ego, en, w, bm):
    n = a_bf.shape[0]
    return pl.pallas_call(
        _layer_body,
        grid=(n // bm,),
        in_specs=[
            pl.BlockSpec((bm, n), lambda i: (i, 0)),
            pl.BlockSpec((n, _LAT), lambda i: (0, 0)),
            pl.BlockSpec((bm, 1), lambda i: (i, 0)),
            pl.BlockSpec((bm, _LAT), lambda i: (i, 0)),
            pl.BlockSpec((bm, 1), lambda i: (i, 0)),
            pl.BlockSpec((_LAT, _LAT), lambda i: (0, 0)),
        ],
        out_specs=[
            pl.BlockSpec((bm, _LAT), lambda i: (i, 0)),
            pl.BlockSpec((bm, _LAT), lambda i: (i, 0)),
        ],
        out_shape=[
            jax.ShapeDtypeStruct((n, _LAT), jnp.float32),
            jax.ShapeDtypeStruct((n, _LAT), jnp.bfloat16),
        ],
    )(a_bf, g, d, ego, en, w)


def _pred_body(l1, l2, l3, d1, d2, d3, out_ref):
    lm = ((l1[...] + l2[...] + l3[...]) * (1.0 / 3.0)).astype(jnp.bfloat16)
    dm = ((d1[...] + d2[...] + d3[...]) * (1.0 / 3.0)).astype(jnp.bfloat16)
    out_ref[...] = jax.lax.dot_general(
        lm, dm, (((1,), (1,)), ((), ())), preferred_element_type=jnp.float32)


def _pred_call(ls, ds, bm):
    lr = ls[0].shape[0]
    dr = ds[0].shape[0]
    return pl.pallas_call(
        _pred_body,
        grid=(lr // bm,),
        in_specs=[pl.BlockSpec((bm, _LAT), lambda i: (i, 0))] * 3
        + [pl.BlockSpec((dr, _LAT), lambda i: (0, 0))] * 3,
        out_specs=pl.BlockSpec((bm, dr), lambda i: (i, 0)),
        out_shape=jax.ShapeDtypeStruct((lr, dr), jnp.float32),
    )(*ls, *ds)


def kernel(A_stack, lnc_sim, dis_sim, miR_sim, W_l, b_l, W_d, b_d, W_m, b_m,
           weight):
    l_num = lnc_sim.shape[0]
    d_num = dis_sim.shape[0]
    n = A_stack.shape[0]

    e_l, n_l = _ego_call(lnc_sim, W_l, b_l, bm=min(512, l_num))
    e_d, n_d = _ego_call(dis_sim, W_d, b_d, bm=min(1024, d_num))
    e_m, n_m = _ego_call(miR_sim, W_m, b_m, bm=min(1024, d_num))
    ego_all = jnp.concatenate([e_l, e_d, e_m], axis=0)
    en = jnp.concatenate([n_l, n_d, n_m], axis=0)

    a_bf, d = _prep_call(A_stack, bm=min(256, n))

    g = _g0_call(ego_all, d, weight, bm=min(2048, n))
    contribs = []
    for _ in range(3):
        contrib, g = _layer_call(a_bf, g, d, ego_all, en, weight,
                                 bm=min(512, n))
        contribs.append(contrib)

    ls = [c[:l_num] for c in contribs]
    ds = [c[l_num:l_num + d_num] for c in contribs]
    return _pred_call(ls, ds, bm=min(1024, l_num))
